# baseline (speedup 1.0000x reference)
import jax
import jax.numpy as jnp
from jax import lax
from jax.experimental import pallas as pl
from jax.experimental.pallas import tpu as pltpu
from jax.experimental.pallas import tpu_sc as plsc

N = 10000
E = 320000
C1 = 128
C2 = 64
NC, NS, L = 2, 16, 16
NW = NC * NS
EWP = 10240
EPAD = NW * EWP
K = 80
CH = EWP // K
TOTCH = EPAD // K
GC = 16
NG0, NG1 = 11, 5
NPAD = 10240
RPT = NPAD // NS


def _deg_body(dst_hbm, w_hbm, out_hbm, dstv, wv, deg):
    c = lax.axis_index("c")
    s = lax.axis_index("s")
    wid = c * NS + s
    pltpu.sync_copy(dst_hbm.at[wid], dstv)
    pltpu.sync_copy(w_hbm.at[wid], wv)
    zero = jnp.zeros((L,), jnp.float32)

    def zbody(i, carry):
        deg[pl.ds(i * L, L)] = zero
        return carry

    lax.fori_loop(0, NPAD // L, zbody, 0)

    def cbody(ci, carry):
        for j in range(K // L):
            dvec = dstv[ci, pl.ds(j * L, L)]
            wvec = wv[ci, pl.ds(j * L, L)]
            plsc.addupdate_scatter(deg, [dvec], wvec)
        return carry

    lax.fori_loop(0, CH, cbody, 0)
    pltpu.sync_copy(deg, out_hbm.at[wid])


_deg_call = pl.kernel(
    _deg_body,
    out_type=jax.ShapeDtypeStruct((NW, NPAD), jnp.float32),
    mesh=plsc.VectorSubcoreMesh(core_axis_name="c", subcore_axis_name="s"),
    compiler_params=pltpu.CompilerParams(needs_layout_passes=False),
    scratch_types=[
        pltpu.VMEM((CH, K), jnp.int32),
        pltpu.VMEM((CH, K), jnp.float32),
        pltpu.VMEM((NPAD,), jnp.float32),
    ],
)


_MASKHI = jnp.int32(-65536)


def _make_agg(C):
    NP = GC // 2

    def body(h_hbm, src_hbm, dst_hbm, w_hbm, out_hbm,
             idx_s, idx_d, wv, b0, b1, sbuf, acc, sga, sgb, ss):
        c = lax.axis_index("c")
        s = lax.axis_index("s")
        ngrp = jnp.where(c == 0, NG0, NG1)
        base_w = jnp.where(c == 0, s * (NG0 * GC), NG0 * GC * NS + s * (NG1 * GC))
        zero = jnp.zeros((L,), jnp.float32)

        def zb(i, carry):
            for r in range(C // L):
                sbuf[i, pl.ds(r * L, L)] = zero
            return carry

        lax.fori_loop(0, K, zb, 0)
        for kk in range(RPT // K):
            pltpu.sync_copy(sbuf, acc.at[pl.ds(s * RPT + kk * K, K)])
        plsc.subcore_barrier()

        def scale(bufb, ci):
            def egrp(g, ecarry):
                wvec = wv[ci, pl.ds(g * L, L)]
                base = g * L
                for l in range(L):
                    wj = wvec[l]
                    for r in range(C // 32):
                        v = bufb[base + l, pl.ds(r * 32, 32)]
                        wd = plsc.bitcast(v, jnp.int32)
                        lo = plsc.bitcast(wd << 16, jnp.float32)
                        hi = plsc.bitcast(wd & _MASKHI, jnp.float32)
                        sbuf[base + l, pl.ds(r * 32, L)] = lo * wj
                        sbuf[base + l, pl.ds(r * 32 + L, L)] = hi * wj
                return ecarry

            lax.fori_loop(0, K // L, egrp, 0)

        def group(g, carry):
            cb = pl.multiple_of(base_w + g * GC, 8)
            pltpu.sync_copy(src_hbm.at[pl.ds(cb, GC)], idx_s)
            pltpu.sync_copy(dst_hbm.at[pl.ds(cb, GC)], idx_d)
            pltpu.sync_copy(w_hbm.at[pl.ds(cb, GC)], wv)
            pltpu.async_copy(h_hbm.at[idx_s.at[0]], b0, sga)

            def pair(p, carry2):
                c0 = p * 2
                pltpu.make_async_copy(h_hbm.at[idx_s.at[c0]], b0, sga).wait()
                pltpu.async_copy(h_hbm.at[idx_s.at[c0 + 1]], b1, sgb)

                @pl.when(p > 0)
                def _():
                    pltpu.make_async_copy(
                        sbuf, acc.at[idx_d.at[c0 - 1]], ss).wait()

                scale(b0, c0)
                pltpu.async_copy(sbuf, acc.at[idx_d.at[c0]], ss, add=True)
                pltpu.make_async_copy(h_hbm.at[idx_s.at[c0 + 1]], b1, sgb).wait()

                @pl.when(p < NP - 1)
                def _():
                    pltpu.async_copy(h_hbm.at[idx_s.at[c0 + 2]], b0, sga)

                pltpu.make_async_copy(sbuf, acc.at[idx_d.at[c0]], ss).wait()
                scale(b1, c0 + 1)
                pltpu.async_copy(sbuf, acc.at[idx_d.at[c0 + 1]], ss, add=True)
                return carry2

            lax.fori_loop(0, NP, pair, 0)
            pltpu.make_async_copy(sbuf, acc.at[idx_d.at[GC - 1]], ss).wait()
            return carry

        lax.fori_loop(0, ngrp, group, 0)
        plsc.subcore_barrier()
        pltpu.sync_copy(acc.at[pl.ds(s * RPT, RPT)],
                        out_hbm.at[c, pl.ds(s * RPT, RPT)])

    return pl.kernel(
        body,
        out_type=jax.ShapeDtypeStruct((NC, NPAD, C), jnp.float32),
        mesh=plsc.VectorSubcoreMesh(core_axis_name="c", subcore_axis_name="s"),
        compiler_params=pltpu.CompilerParams(
            needs_layout_passes=False, use_tc_tiling_on_sc=False),
        scratch_types=[
            pltpu.VMEM((GC, K), jnp.int32),
            pltpu.VMEM((GC, K), jnp.int32),
            pltpu.VMEM((GC, K), jnp.float32),
            pltpu.VMEM((K, C), jnp.bfloat16),
            pltpu.VMEM((K, C), jnp.bfloat16),
            pltpu.VMEM((K, C), jnp.float32),
            pltpu.VMEM_SHARED((NPAD, C), jnp.float32),
            pltpu.SemaphoreType.DMA,
            pltpu.SemaphoreType.DMA,
            pltpu.SemaphoreType.DMA,
        ],
    )


_agg_c1 = _make_agg(C1)
_agg_c2 = _make_agg(C2)


def _dinv_from_parts(degp):
    deg = 1.0 + jnp.sum(degp[:, :N], axis=0)
    return jnp.where(deg > 0, lax.rsqrt(deg), 0.0)


def _tc_h1_body(x_ref, w1_ref, degp_ref, h1p_ref):
    dinv = _dinv_from_parts(degp_ref[...])
    h = jnp.dot(x_ref[...], w1_ref[...], preferred_element_type=jnp.float32)
    h1p_ref[...] = h * dinv[:, None]


def _tc_mid_body(agg_ref, h1p_ref, degp_ref, b1_ref, w2_ref, h2p_ref):
    dinv = _dinv_from_parts(degp_ref[...])[:, None]
    t = dinv * (agg_ref[0, :N] + agg_ref[1, :N] + h1p_ref[...]) + b1_ref[...]
    z = jnp.maximum(t, 0.0)
    h2 = jnp.dot(z, w2_ref[...], preferred_element_type=jnp.float32)
    h2p_ref[...] = h2 * dinv


def _tc_out_body(agg_ref, h2p_ref, degp_ref, b2_ref, o_ref):
    dinv = _dinv_from_parts(degp_ref[...])[:, None]
    t = dinv * (agg_ref[0, :N] + agg_ref[1, :N] + h2p_ref[...])
    o_ref[...] = t[:, :40] + b2_ref[...]


def _permcast(h, C):
    return (h.reshape(N, C // 32, 2, L).transpose(0, 1, 3, 2)
            .reshape(N, C).astype(jnp.bfloat16))


def kernel(x, edge_index, edge_atr, W1, b1, W2, b2):
    npad = EPAD - E
    srcf = jnp.pad(edge_index[0], (0, npad))
    dstf = jnp.pad(edge_index[1], (0, npad))
    wf = jnp.pad(edge_atr, (0, npad))
    src = srcf.reshape(TOTCH, K)
    dst = dstf.reshape(TOTCH, K)
    w3 = wf.reshape(TOTCH, K)

    degp = _deg_call(dstf.reshape(NW, CH, K), wf.reshape(NW, CH, K))

    h1p = pl.pallas_call(
        _tc_h1_body,
        out_shape=jax.ShapeDtypeStruct((N, C1), jnp.float32),
    )(x, W1, degp)

    agg1 = _agg_c1(_permcast(h1p, C1), src, dst, w3)

    W2p = jnp.pad(W2, ((0, 0), (0, C2 - W2.shape[1])))
    h2p = pl.pallas_call(
        _tc_mid_body,
        out_shape=jax.ShapeDtypeStruct((N, C2), jnp.float32),
    )(agg1, h1p, degp, b1.reshape(1, C1), W2p)

    agg2 = _agg_c2(_permcast(h2p, C2), src, dst, w3)

    out = pl.pallas_call(
        _tc_out_body,
        out_shape=jax.ShapeDtypeStruct((N, 40), jnp.float32),
    )(agg2, h2p, degp, b2.reshape(1, 40))
    return out


# speedup vs baseline: 1.1855x; 1.1855x over previous
import jax
import jax.numpy as jnp
from jax import lax
from jax.experimental import pallas as pl
from jax.experimental.pallas import tpu as pltpu
from jax.experimental.pallas import tpu_sc as plsc

N = 10000
E = 320000
C1 = 128
C2 = 64
NC, NS, L = 2, 16, 16
NW = NC * NS
EWP = 10240
EPAD = NW * EWP
K = 64
CH = EWP // K
TOTCH = EPAD // K
GC = 16
NG0, NG1 = 13, 7
NPAD = 10240
RPT = NPAD // NS


def _deg_body(dst_hbm, w_hbm, out_hbm, dstv, wv, deg):
    c = lax.axis_index("c")
    s = lax.axis_index("s")
    wid = c * NS + s
    pltpu.sync_copy(dst_hbm.at[wid], dstv)
    pltpu.sync_copy(w_hbm.at[wid], wv)
    zero = jnp.zeros((L,), jnp.float32)

    def zbody(i, carry):
        deg[pl.ds(i * L, L)] = zero
        return carry

    lax.fori_loop(0, NPAD // L, zbody, 0)

    def cbody(ci, carry):
        for j in range(K // L):
            dvec = dstv[ci, pl.ds(j * L, L)]
            wvec = wv[ci, pl.ds(j * L, L)]
            plsc.addupdate_scatter(deg, [dvec], wvec)
        return carry

    lax.fori_loop(0, CH, cbody, 0)
    pltpu.sync_copy(deg, out_hbm.at[wid])


_deg_call = pl.kernel(
    _deg_body,
    out_type=jax.ShapeDtypeStruct((NW, NPAD), jnp.float32),
    mesh=plsc.VectorSubcoreMesh(core_axis_name="c", subcore_axis_name="s"),
    compiler_params=pltpu.CompilerParams(needs_layout_passes=False),
    scratch_types=[
        pltpu.VMEM((CH, K), jnp.int32),
        pltpu.VMEM((CH, K), jnp.float32),
        pltpu.VMEM((NPAD,), jnp.float32),
    ],
)


_MASKHI = jnp.int32(-65536)


def _make_agg(C):
    NP = GC // 2

    def body(h_hbm, src_hbm, dst_hbm, w_hbm, out_hbm,
             idx_s, idx_d, wv, b0, b1, s0, s1, acc, sga, sgb, ss0, ss1):
        c = lax.axis_index("c")
        s = lax.axis_index("s")
        ngrp = jnp.where(c == 0, NG0, NG1)
        base_w = jnp.where(c == 0, s * (NG0 * GC), NG0 * GC * NS + s * (NG1 * GC))
        zero = jnp.zeros((L,), jnp.float32)

        def zb(i, carry):
            for r in range(C // L):
                s0[i, pl.ds(r * L, L)] = zero
            return carry

        lax.fori_loop(0, K, zb, 0)
        for kk in range(RPT // K):
            pltpu.sync_copy(s0, acc.at[pl.ds(s * RPT + kk * K, K)])
        plsc.subcore_barrier()

        def scale(bufb, sbuf, ci):
            def egrp(g, ecarry):
                wvec = wv[ci, pl.ds(g * L, L)]
                base = g * L
                for l in range(L):
                    wj = wvec[l]
                    for r in range(C // 32):
                        v = bufb[base + l, pl.ds(r * 32, 32)]
                        wd = plsc.bitcast(v, jnp.int32)
                        lo = plsc.bitcast(wd << 16, jnp.float32)
                        hi = plsc.bitcast(wd & _MASKHI, jnp.float32)
                        sbuf[base + l, pl.ds(r * 32, L)] = lo * wj
                        sbuf[base + l, pl.ds(r * 32 + L, L)] = hi * wj
                return ecarry

            lax.fori_loop(0, K // L, egrp, 0)

        def group(g, carry):
            cb = pl.multiple_of(base_w + g * GC, 8)
            pltpu.sync_copy(src_hbm.at[pl.ds(cb, GC)], idx_s)
            pltpu.sync_copy(dst_hbm.at[pl.ds(cb, GC)], idx_d)
            pltpu.sync_copy(w_hbm.at[pl.ds(cb, GC)], wv)
            pltpu.async_copy(h_hbm.at[idx_s.at[0]], b0, sga)

            def pair(p, carry2):
                c0 = p * 2
                pltpu.make_async_copy(h_hbm.at[idx_s.at[c0]], b0, sga).wait()
                pltpu.async_copy(h_hbm.at[idx_s.at[c0 + 1]], b1, sgb)

                @pl.when(p > 0)
                def _():
                    pltpu.make_async_copy(
                        s0, acc.at[idx_d.at[c0 - 2]], ss0).wait()

                scale(b0, s0, c0)
                pltpu.async_copy(s0, acc.at[idx_d.at[c0]], ss0, add=True)
                pltpu.make_async_copy(h_hbm.at[idx_s.at[c0 + 1]], b1, sgb).wait()

                @pl.when(p < NP - 1)
                def _():
                    pltpu.async_copy(h_hbm.at[idx_s.at[c0 + 2]], b0, sga)

                @pl.when(p > 0)
                def _():
                    pltpu.make_async_copy(
                        s1, acc.at[idx_d.at[c0 - 1]], ss1).wait()

                scale(b1, s1, c0 + 1)
                pltpu.async_copy(s1, acc.at[idx_d.at[c0 + 1]], ss1, add=True)
                return carry2

            lax.fori_loop(0, NP, pair, 0)
            pltpu.make_async_copy(s0, acc.at[idx_d.at[GC - 2]], ss0).wait()
            pltpu.make_async_copy(s1, acc.at[idx_d.at[GC - 1]], ss1).wait()
            return carry

        lax.fori_loop(0, ngrp, group, 0)
        plsc.subcore_barrier()
        pltpu.sync_copy(acc.at[pl.ds(s * RPT, RPT)],
                        out_hbm.at[c, pl.ds(s * RPT, RPT)])

    return pl.kernel(
        body,
        out_type=jax.ShapeDtypeStruct((NC, NPAD, C), jnp.float32),
        mesh=plsc.VectorSubcoreMesh(core_axis_name="c", subcore_axis_name="s"),
        compiler_params=pltpu.CompilerParams(
            needs_layout_passes=False, use_tc_tiling_on_sc=False),
        scratch_types=[
            pltpu.VMEM((GC, K), jnp.int32),
            pltpu.VMEM((GC, K), jnp.int32),
            pltpu.VMEM((GC, K), jnp.float32),
            pltpu.VMEM((K, C), jnp.bfloat16),
            pltpu.VMEM((K, C), jnp.bfloat16),
            pltpu.VMEM((K, C), jnp.float32),
            pltpu.VMEM((K, C), jnp.float32),
            pltpu.VMEM_SHARED((NPAD, C), jnp.float32),
            pltpu.SemaphoreType.DMA,
            pltpu.SemaphoreType.DMA,
            pltpu.SemaphoreType.DMA,
            pltpu.SemaphoreType.DMA,
        ],
    )


_agg_c1 = _make_agg(C1)
_agg_c2 = _make_agg(C2)


def _dinv_from_parts(degp):
    deg = 1.0 + jnp.sum(degp[:, :N], axis=0)
    return jnp.where(deg > 0, lax.rsqrt(deg), 0.0)


def _tc_h1_body(x_ref, w1_ref, degp_ref, h1p_ref):
    dinv = _dinv_from_parts(degp_ref[...])
    h = jnp.dot(x_ref[...], w1_ref[...], preferred_element_type=jnp.float32)
    h1p_ref[...] = h * dinv[:, None]


def _tc_mid_body(agg_ref, h1p_ref, degp_ref, b1_ref, w2_ref, h2p_ref):
    dinv = _dinv_from_parts(degp_ref[...])[:, None]
    t = dinv * (agg_ref[0, :N] + agg_ref[1, :N] + h1p_ref[...]) + b1_ref[...]
    z = jnp.maximum(t, 0.0)
    h2 = jnp.dot(z, w2_ref[...], preferred_element_type=jnp.float32)
    h2p_ref[...] = h2 * dinv


def _tc_out_body(agg_ref, h2p_ref, degp_ref, b2_ref, o_ref):
    dinv = _dinv_from_parts(degp_ref[...])[:, None]
    t = dinv * (agg_ref[0, :N] + agg_ref[1, :N] + h2p_ref[...])
    o_ref[...] = t[:, :40] + b2_ref[...]


def _permcast(h, C):
    return (h.reshape(N, C // 32, 2, L).transpose(0, 1, 3, 2)
            .reshape(N, C).astype(jnp.bfloat16))


def kernel(x, edge_index, edge_atr, W1, b1, W2, b2):
    npad = EPAD - E
    srcf = jnp.pad(edge_index[0], (0, npad))
    dstf = jnp.pad(edge_index[1], (0, npad))
    wf = jnp.pad(edge_atr, (0, npad))
    src = srcf.reshape(TOTCH, K)
    dst = dstf.reshape(TOTCH, K)
    w3 = wf.reshape(TOTCH, K)

    degp = _deg_call(dstf.reshape(NW, CH, K), wf.reshape(NW, CH, K))

    h1p = pl.pallas_call(
        _tc_h1_body,
        out_shape=jax.ShapeDtypeStruct((N, C1), jnp.float32),
    )(x, W1, degp)

    agg1 = _agg_c1(_permcast(h1p, C1), src, dst, w3)

    W2p = jnp.pad(W2, ((0, 0), (0, C2 - W2.shape[1])))
    h2p = pl.pallas_call(
        _tc_mid_body,
        out_shape=jax.ShapeDtypeStruct((N, C2), jnp.float32),
    )(agg1, h1p, degp, b1.reshape(1, C1), W2p)

    agg2 = _agg_c2(_permcast(h2p, C2), src, dst, w3)

    out = pl.pallas_call(
        _tc_out_body,
        out_shape=jax.ShapeDtypeStruct((N, 40), jnp.float32),
    )(agg2, h2p, degp, b2.reshape(1, 40))
    return out


# speedup vs baseline: 1.2332x; 1.0403x over previous
import jax
import jax.numpy as jnp
from jax import lax
from jax.experimental import pallas as pl
from jax.experimental.pallas import tpu as pltpu
from jax.experimental.pallas import tpu_sc as plsc

N = 10000
E = 320000
C1 = 128
C2 = 64
NC, NS, L = 2, 16, 16
NW = NC * NS
EWP = 10240
EPAD = NW * EWP
K = 64
CH = EWP // K
TOTCH = EPAD // K
GC = 16
NG0, NG1 = 13, 7
NPAD = 10240
RPT = NPAD // NS


def _deg_body(dst_hbm, w_hbm, out_hbm, dstv, wv, deg):
    c = lax.axis_index("c")
    s = lax.axis_index("s")
    wid = c * NS + s
    pltpu.sync_copy(dst_hbm.at[wid], dstv)
    pltpu.sync_copy(w_hbm.at[wid], wv)
    zero = jnp.zeros((L,), jnp.float32)

    def zbody(i, carry):
        deg[pl.ds(i * L, L)] = zero
        return carry

    lax.fori_loop(0, NPAD // L, zbody, 0)

    def cbody(ci, carry):
        for j in range(K // L):
            dvec = dstv[ci, pl.ds(j * L, L)]
            wvec = wv[ci, pl.ds(j * L, L)]
            plsc.addupdate_scatter(deg, [dvec], wvec)
        return carry

    lax.fori_loop(0, CH, cbody, 0)
    pltpu.sync_copy(deg, out_hbm.at[wid])


_deg_call = pl.kernel(
    _deg_body,
    out_type=jax.ShapeDtypeStruct((NW, NPAD), jnp.float32),
    mesh=plsc.VectorSubcoreMesh(core_axis_name="c", subcore_axis_name="s"),
    compiler_params=pltpu.CompilerParams(needs_layout_passes=False),
    scratch_types=[
        pltpu.VMEM((CH, K), jnp.int32),
        pltpu.VMEM((CH, K), jnp.float32),
        pltpu.VMEM((NPAD,), jnp.float32),
    ],
)


_MASKHI = jnp.int32(-65536)


def _make_agg(C):
    NP = GC // 2

    def body(h_hbm, src_hbm, dst_hbm, w_hbm, out_hbm,
             idx_s, idx_d, wv, b0, b1, s0, s1, acc, sga, sgb, ss0, ss1, stg):
        c = lax.axis_index("c")
        s = lax.axis_index("s")
        ngrp = jnp.where(c == 0, NG0, NG1)
        base_w = jnp.where(c == 0, s * (NG0 * GC), NG0 * GC * NS + s * (NG1 * GC))
        zero = jnp.zeros((L,), jnp.float32)

        def zb(i, carry):
            for r in range(C // L):
                s0[i, pl.ds(r * L, L)] = zero
            return carry

        lax.fori_loop(0, K, zb, 0)
        for kk in range(RPT // K):
            pltpu.sync_copy(s0, acc.at[pl.ds(s * RPT + kk * K, K)])
        plsc.subcore_barrier()

        def scale(bufb, sbuf, ci):
            def egrp(g, ecarry):
                wvec = wv[ci, pl.ds(g * L, L)]
                base = g * L
                for l in range(L):
                    wj = wvec[l]
                    for r in range(C // 32):
                        v = bufb[base + l, pl.ds(r * 32, 32)]
                        wd = plsc.bitcast(v, jnp.int32)
                        lo = plsc.bitcast(wd << 16, jnp.float32)
                        hi = plsc.bitcast(wd & _MASKHI, jnp.float32)
                        sbuf[base + l, pl.ds(r * 32, L)] = lo * wj
                        sbuf[base + l, pl.ds(r * 32 + L, L)] = hi * wj
                return ecarry

            lax.fori_loop(0, K // L, egrp, 0)

        def group(g, carry):
            cb = pl.multiple_of(base_w + g * GC, 8)
            d1 = pltpu.async_copy(src_hbm.at[pl.ds(cb, GC)], idx_s, stg)
            d2 = pltpu.async_copy(dst_hbm.at[pl.ds(cb, GC)], idx_d, stg)
            d3 = pltpu.async_copy(w_hbm.at[pl.ds(cb, GC)], wv, stg)
            d1.wait()
            d2.wait()
            d3.wait()
            pltpu.async_copy(h_hbm.at[idx_s.at[0]], b0, sga)

            def pair(p, carry2):
                c0 = p * 2
                pltpu.make_async_copy(h_hbm.at[idx_s.at[c0]], b0, sga).wait()
                pltpu.async_copy(h_hbm.at[idx_s.at[c0 + 1]], b1, sgb)

                @pl.when(p > 0)
                def _():
                    pltpu.make_async_copy(
                        s0, acc.at[idx_d.at[c0 - 2]], ss0).wait()

                scale(b0, s0, c0)
                pltpu.async_copy(s0, acc.at[idx_d.at[c0]], ss0, add=True)
                pltpu.make_async_copy(h_hbm.at[idx_s.at[c0 + 1]], b1, sgb).wait()

                @pl.when(p < NP - 1)
                def _():
                    pltpu.async_copy(h_hbm.at[idx_s.at[c0 + 2]], b0, sga)

                @pl.when(p > 0)
                def _():
                    pltpu.make_async_copy(
                        s1, acc.at[idx_d.at[c0 - 1]], ss1).wait()

                scale(b1, s1, c0 + 1)
                pltpu.async_copy(s1, acc.at[idx_d.at[c0 + 1]], ss1, add=True)
                return carry2

            lax.fori_loop(0, NP, pair, 0)
            pltpu.make_async_copy(s0, acc.at[idx_d.at[GC - 2]], ss0).wait()
            pltpu.make_async_copy(s1, acc.at[idx_d.at[GC - 1]], ss1).wait()
            return carry

        lax.fori_loop(0, ngrp, group, 0)
        plsc.subcore_barrier()
        pltpu.sync_copy(acc.at[pl.ds(s * RPT, RPT)],
                        out_hbm.at[c, pl.ds(s * RPT, RPT)])

    return pl.kernel(
        body,
        out_type=jax.ShapeDtypeStruct((NC, NPAD, C), jnp.float32),
        mesh=plsc.VectorSubcoreMesh(core_axis_name="c", subcore_axis_name="s"),
        compiler_params=pltpu.CompilerParams(
            needs_layout_passes=False, use_tc_tiling_on_sc=False),
        scratch_types=[
            pltpu.VMEM((GC, K), jnp.int32),
            pltpu.VMEM((GC, K), jnp.int32),
            pltpu.VMEM((GC, K), jnp.float32),
            pltpu.VMEM((K, C), jnp.bfloat16),
            pltpu.VMEM((K, C), jnp.bfloat16),
            pltpu.VMEM((K, C), jnp.float32),
            pltpu.VMEM((K, C), jnp.float32),
            pltpu.VMEM_SHARED((NPAD, C), jnp.float32),
            pltpu.SemaphoreType.DMA,
            pltpu.SemaphoreType.DMA,
            pltpu.SemaphoreType.DMA,
            pltpu.SemaphoreType.DMA,
            pltpu.SemaphoreType.DMA,
        ],
    )


_agg_c1 = _make_agg(C1)
_agg_c2 = _make_agg(C2)


def _dinv_from_parts(degp):
    deg = 1.0 + jnp.sum(degp[:, :N], axis=0)
    return jnp.where(deg > 0, lax.rsqrt(deg), 0.0)


def _tc_h1_body(x_ref, w1_ref, degp_ref, h1p_ref):
    dinv = _dinv_from_parts(degp_ref[...])
    h = jnp.dot(x_ref[...], w1_ref[...], preferred_element_type=jnp.float32)
    h1p_ref[...] = h * dinv[:, None]


def _tc_mid_body(agg_ref, h1p_ref, degp_ref, b1_ref, w2_ref, h2p_ref):
    dinv = _dinv_from_parts(degp_ref[...])[:, None]
    t = dinv * (agg_ref[0, :N] + agg_ref[1, :N] + h1p_ref[...]) + b1_ref[...]
    z = jnp.maximum(t, 0.0)
    h2 = jnp.dot(z, w2_ref[...], preferred_element_type=jnp.float32)
    h2p_ref[...] = h2 * dinv


def _tc_out_body(agg_ref, h2p_ref, degp_ref, b2_ref, o_ref):
    dinv = _dinv_from_parts(degp_ref[...])[:, None]
    t = dinv * (agg_ref[0, :N] + agg_ref[1, :N] + h2p_ref[...])
    o_ref[...] = t[:, :40] + b2_ref[...]


def _permcast(h, C):
    return (h.reshape(N, C // 32, 2, L).transpose(0, 1, 3, 2)
            .reshape(N, C).astype(jnp.bfloat16))


def kernel(x, edge_index, edge_atr, W1, b1, W2, b2):
    npad = EPAD - E
    srcf = jnp.pad(edge_index[0], (0, npad))
    dstf = jnp.pad(edge_index[1], (0, npad))
    wf = jnp.pad(edge_atr, (0, npad))
    src = srcf.reshape(TOTCH, K)
    dst = dstf.reshape(TOTCH, K)
    w3 = wf.reshape(TOTCH, K)

    degp = _deg_call(dstf.reshape(NW, CH, K), wf.reshape(NW, CH, K))

    h1p = pl.pallas_call(
        _tc_h1_body,
        out_shape=jax.ShapeDtypeStruct((N, C1), jnp.float32),
    )(x, W1, degp)

    agg1 = _agg_c1(_permcast(h1p, C1), src, dst, w3)

    W2p = jnp.pad(W2, ((0, 0), (0, C2 - W2.shape[1])))
    h2p = pl.pallas_call(
        _tc_mid_body,
        out_shape=jax.ShapeDtypeStruct((N, C2), jnp.float32),
    )(agg1, h1p, degp, b1.reshape(1, C1), W2p)

    agg2 = _agg_c2(_permcast(h2p, C2), src, dst, w3)

    out = pl.pallas_call(
        _tc_out_body,
        out_shape=jax.ShapeDtypeStruct((N, 40), jnp.float32),
    )(agg2, h2p, degp, b2.reshape(1, 40))
    return out


# speedup vs baseline: 1.2621x; 1.0234x over previous
import jax
import jax.numpy as jnp
from jax import lax
from jax.experimental import pallas as pl
from jax.experimental.pallas import tpu as pltpu
from jax.experimental.pallas import tpu_sc as plsc

N = 10000
E = 320000
C1 = 128
C2 = 64
NC, NS, L = 2, 16, 16
NW = NC * NS
EWP = 10240
EPAD = NW * EWP
K = 64
CH = EWP // K
TOTCH = EPAD // K
GC = 16
NG0, NG1 = 13, 7
NPAD = 10240
RPT = NPAD // NS


def _deg_body(dst_hbm, w_hbm, out_hbm, dstv, wv, deg):
    c = lax.axis_index("c")
    s = lax.axis_index("s")
    wid = c * NS + s
    pltpu.sync_copy(dst_hbm.at[wid], dstv)
    pltpu.sync_copy(w_hbm.at[wid], wv)
    zero = jnp.zeros((L,), jnp.float32)

    def zbody(i, carry):
        deg[pl.ds(i * L, L)] = zero
        return carry

    lax.fori_loop(0, NPAD // L, zbody, 0)

    def cbody(ci, carry):
        for j in range(K // L):
            dvec = dstv[ci, pl.ds(j * L, L)]
            wvec = wv[ci, pl.ds(j * L, L)]
            plsc.addupdate_scatter(deg, [dvec], wvec)
        return carry

    lax.fori_loop(0, CH, cbody, 0)
    pltpu.sync_copy(deg, out_hbm.at[wid])


_deg_call = pl.kernel(
    _deg_body,
    out_type=jax.ShapeDtypeStruct((NW, NPAD), jnp.float32),
    mesh=plsc.VectorSubcoreMesh(core_axis_name="c", subcore_axis_name="s"),
    compiler_params=pltpu.CompilerParams(needs_layout_passes=False),
    scratch_types=[
        pltpu.VMEM((CH, K), jnp.int32),
        pltpu.VMEM((CH, K), jnp.float32),
        pltpu.VMEM((NPAD,), jnp.float32),
    ],
)


_MASKHI = jnp.int32(-65536)


def _make_agg(C):
    NP = GC // 2

    def body(h_hbm, src_hbm, dst_hbm, w_hbm, out_hbm,
             idx_s, idx_d, wv, b0, b1, s0, s1, acc,
             sga, sgb, ss0, ss1, stg, stgd):
        c = lax.axis_index("c")
        s = lax.axis_index("s")
        ngrp = jnp.where(c == 0, NG0, NG1)
        base_w = jnp.where(c == 0, s * (NG0 * GC), NG0 * GC * NS + s * (NG1 * GC))
        zero = jnp.zeros((L,), jnp.float32)

        def zb(i, carry):
            for r in range(C // L):
                s0[i, pl.ds(r * L, L)] = zero
            return carry

        lax.fori_loop(0, K, zb, 0)
        for kk in range(RPT // K):
            pltpu.sync_copy(s0, acc.at[pl.ds(s * RPT + kk * K, K)])
        plsc.subcore_barrier()

        def scale(bufb, sbuf, ci, sel):
            def egrp(g, ecarry):
                wvec = wv[sel, ci, pl.ds(g * L, L)]
                base = g * L
                for l in range(L):
                    wj = wvec[l]
                    for r in range(C // 32):
                        v = bufb[base + l, pl.ds(r * 32, 32)]
                        wd = plsc.bitcast(v, jnp.int32)
                        lo = plsc.bitcast(wd << 16, jnp.float32)
                        hi = plsc.bitcast(wd & _MASKHI, jnp.float32)
                        sbuf[base + l, pl.ds(r * 32, L)] = lo * wj
                        sbuf[base + l, pl.ds(r * 32 + L, L)] = hi * wj
                return ecarry

            lax.fori_loop(0, K // L, egrp, 0)

        cb0 = pl.multiple_of(base_w, 8)
        pltpu.async_copy(src_hbm.at[pl.ds(cb0, GC)], idx_s.at[0], stg)
        pltpu.async_copy(w_hbm.at[pl.ds(cb0, GC)], wv.at[0], stg)

        def group(g, carry):
            sel = g % 2
            cb = pl.multiple_of(base_w + g * GC, 8)
            pltpu.make_async_copy(
                src_hbm.at[pl.ds(cb, GC)], idx_s.at[sel], stg).wait()
            pltpu.make_async_copy(
                w_hbm.at[pl.ds(cb, GC)], wv.at[sel], stg).wait()
            pltpu.async_copy(dst_hbm.at[pl.ds(cb, GC)], idx_d, stgd)

            @pl.when(g + 1 < ngrp)
            def _():
                cb2 = pl.multiple_of(base_w + (g + 1) * GC, 8)
                pltpu.async_copy(src_hbm.at[pl.ds(cb2, GC)], idx_s.at[1 - sel], stg)
                pltpu.async_copy(w_hbm.at[pl.ds(cb2, GC)], wv.at[1 - sel], stg)

            pltpu.async_copy(h_hbm.at[idx_s.at[sel, 0]], b0, sga)

            def pair(p, carry2):
                c0 = p * 2
                pltpu.make_async_copy(h_hbm.at[idx_s.at[sel, c0]], b0, sga).wait()
                pltpu.async_copy(h_hbm.at[idx_s.at[sel, c0 + 1]], b1, sgb)

                @pl.when(p == 0)
                def _():
                    pltpu.make_async_copy(
                        dst_hbm.at[pl.ds(cb, GC)], idx_d, stgd).wait()

                @pl.when(p > 0)
                def _():
                    pltpu.make_async_copy(
                        s0, acc.at[idx_d.at[c0 - 2]], ss0).wait()

                scale(b0, s0, c0, sel)
                pltpu.async_copy(s0, acc.at[idx_d.at[c0]], ss0, add=True)
                pltpu.make_async_copy(h_hbm.at[idx_s.at[sel, c0 + 1]], b1, sgb).wait()

                @pl.when(p < NP - 1)
                def _():
                    pltpu.async_copy(h_hbm.at[idx_s.at[sel, c0 + 2]], b0, sga)

                @pl.when(p > 0)
                def _():
                    pltpu.make_async_copy(
                        s1, acc.at[idx_d.at[c0 - 1]], ss1).wait()

                scale(b1, s1, c0 + 1, sel)
                pltpu.async_copy(s1, acc.at[idx_d.at[c0 + 1]], ss1, add=True)
                return carry2

            lax.fori_loop(0, NP, pair, 0)
            pltpu.make_async_copy(s0, acc.at[idx_d.at[GC - 2]], ss0).wait()
            pltpu.make_async_copy(s1, acc.at[idx_d.at[GC - 1]], ss1).wait()
            return carry

        lax.fori_loop(0, ngrp, group, 0)
        plsc.subcore_barrier()
        pltpu.sync_copy(acc.at[pl.ds(s * RPT, RPT)],
                        out_hbm.at[c, pl.ds(s * RPT, RPT)])

    return pl.kernel(
        body,
        out_type=jax.ShapeDtypeStruct((NC, NPAD, C), jnp.float32),
        mesh=plsc.VectorSubcoreMesh(core_axis_name="c", subcore_axis_name="s"),
        compiler_params=pltpu.CompilerParams(
            needs_layout_passes=False, use_tc_tiling_on_sc=False),
        scratch_types=[
            pltpu.VMEM((2, GC, K), jnp.int32),
            pltpu.VMEM((GC, K), jnp.int32),
            pltpu.VMEM((2, GC, K), jnp.float32),
            pltpu.VMEM((K, C), jnp.bfloat16),
            pltpu.VMEM((K, C), jnp.bfloat16),
            pltpu.VMEM((K, C), jnp.float32),
            pltpu.VMEM((K, C), jnp.float32),
            pltpu.VMEM_SHARED((NPAD, C), jnp.float32),
            pltpu.SemaphoreType.DMA,
            pltpu.SemaphoreType.DMA,
            pltpu.SemaphoreType.DMA,
            pltpu.SemaphoreType.DMA,
            pltpu.SemaphoreType.DMA,
            pltpu.SemaphoreType.DMA,
        ],
    )


_agg_c1 = _make_agg(C1)
_agg_c2 = _make_agg(C2)


def _dinv_from_parts(degp):
    deg = 1.0 + jnp.sum(degp[:, :N], axis=0)
    return jnp.where(deg > 0, lax.rsqrt(deg), 0.0)


def _tc_h1_body(x_ref, w1_ref, degp_ref, h1p_ref):
    dinv = _dinv_from_parts(degp_ref[...])
    h = jnp.dot(x_ref[...], w1_ref[...], preferred_element_type=jnp.float32)
    h1p_ref[...] = h * dinv[:, None]


def _tc_mid_body(agg_ref, h1p_ref, degp_ref, b1_ref, w2_ref, h2p_ref):
    dinv = _dinv_from_parts(degp_ref[...])[:, None]
    t = dinv * (agg_ref[0, :N] + agg_ref[1, :N] + h1p_ref[...]) + b1_ref[...]
    z = jnp.maximum(t, 0.0)
    h2 = jnp.dot(z, w2_ref[...], preferred_element_type=jnp.float32)
    h2p_ref[...] = h2 * dinv


def _tc_out_body(agg_ref, h2p_ref, degp_ref, b2_ref, o_ref):
    dinv = _dinv_from_parts(degp_ref[...])[:, None]
    t = dinv * (agg_ref[0, :N] + agg_ref[1, :N] + h2p_ref[...])
    o_ref[...] = t[:, :40] + b2_ref[...]


def _permcast(h, C):
    return (h.reshape(N, C // 32, 2, L).transpose(0, 1, 3, 2)
            .reshape(N, C).astype(jnp.bfloat16))


def kernel(x, edge_index, edge_atr, W1, b1, W2, b2):
    npad = EPAD - E
    srcf = jnp.pad(edge_index[0], (0, npad))
    dstf = jnp.pad(edge_index[1], (0, npad))
    wf = jnp.pad(edge_atr, (0, npad))
    src = srcf.reshape(TOTCH, K)
    dst = dstf.reshape(TOTCH, K)
    w3 = wf.reshape(TOTCH, K)

    degp = _deg_call(dstf.reshape(NW, CH, K), wf.reshape(NW, CH, K))

    h1p = pl.pallas_call(
        _tc_h1_body,
        out_shape=jax.ShapeDtypeStruct((N, C1), jnp.float32),
    )(x, W1, degp)

    agg1 = _agg_c1(_permcast(h1p, C1), src, dst, w3)

    W2p = jnp.pad(W2, ((0, 0), (0, C2 - W2.shape[1])))
    h2p = pl.pallas_call(
        _tc_mid_body,
        out_shape=jax.ShapeDtypeStruct((N, C2), jnp.float32),
    )(agg1, h1p, degp, b1.reshape(1, C1), W2p)

    agg2 = _agg_c2(_permcast(h2p, C2), src, dst, w3)

    out = pl.pallas_call(
        _tc_out_body,
        out_shape=jax.ShapeDtypeStruct((N, 40), jnp.float32),
    )(agg2, h2p, degp, b2.reshape(1, 40))
    return out
